# baseline (speedup 1.0000x reference)
import functools

import jax
import jax.numpy as jnp
from jax import lax
from jax.experimental import pallas as pl
from jax.experimental.pallas import tpu as pltpu
from jax.experimental.pallas import tpu_sc as plsc

B = 16384
L = 200
D = 32
NW = 32
BPW = B // NW
CH = 40
NCH = L // CH


def _pool_body(idxt_hbm, table_hbm, pooled_hbm,
               idx_a, idx_b, acc0, acc1, sem_i, sem0, sem1):
    wid = lax.axis_index("s") * 2 + lax.axis_index("c")
    base = wid * BPW
    idx_bufs = (idx_a, idx_b)
    sems = (sem_i, sem_i)

    def idx_fetch(c, buf):
        return pltpu.async_copy(
            idxt_hbm.at[pl.ds(c * CH, CH), pl.ds(base, BPW)], buf, sem_i)

    idx_fetch(0, idx_a).wait()
    fetch1 = idx_fetch(1, idx_b)

    pltpu.async_copy(table_hbm.at[idx_a.at[0]], acc0, sem0)
    pltpu.async_copy(table_hbm.at[idx_a.at[1]], acc1, sem1)

    def make_pair_body(idx_buf):
        def pair_body(k, _):
            row0 = idx_buf.at[2 * k]
            row1 = idx_buf.at[2 * k + 1]
            pltpu.make_async_copy(table_hbm.at[row0], acc0, sem0).wait()
            pltpu.async_copy(table_hbm.at[row0], acc0, sem0, add=True)
            pltpu.make_async_copy(table_hbm.at[row1], acc1, sem1).wait()
            pltpu.async_copy(table_hbm.at[row1], acc1, sem1, add=True)
            return 0
        return pair_body

    lax.fori_loop(1, CH // 2, make_pair_body(idx_a), 0)

    pending = fetch1
    for c in range(1, NCH):
        buf = idx_bufs[c % 2]
        pending.wait()
        if c + 1 < NCH:
            pending = idx_fetch(c + 1, idx_bufs[(c + 1) % 2])
        lax.fori_loop(0, CH // 2, make_pair_body(buf), 0)

    pltpu.make_async_copy(table_hbm.at[idx_a.at[0]], acc0, sem0).wait()
    pltpu.make_async_copy(table_hbm.at[idx_a.at[1]], acc1, sem1).wait()

    def comb_body(r, _):
        acc0[r, pl.ds(0, 16)] = acc0[r, pl.ds(0, 16)] + acc1[r, pl.ds(0, 16)]
        acc0[r, pl.ds(16, 16)] = acc0[r, pl.ds(16, 16)] + acc1[r, pl.ds(16, 16)]
        return 0

    lax.fori_loop(0, BPW, comb_body, 0)
    pltpu.sync_copy(acc0, pooled_hbm.at[pl.ds(base, BPW)])


_pool = functools.partial(
    pl.kernel,
    mesh=plsc.VectorSubcoreMesh(core_axis_name="c", subcore_axis_name="s"),
    compiler_params=pltpu.CompilerParams(use_tc_tiling_on_sc=False),
    out_type=jax.ShapeDtypeStruct((B, D), jnp.float32),
    scratch_types=[
        pltpu.VMEM((CH, BPW), jnp.int32),
        pltpu.VMEM((CH, BPW), jnp.int32),
        pltpu.VMEM((BPW, D), jnp.float32),
        pltpu.VMEM((BPW, D), jnp.float32),
        pltpu.SemaphoreType.DMA,
        pltpu.SemaphoreType.DMA,
        pltpu.SemaphoreType.DMA,
    ],
)(_pool_body)


_TR_BLK = 2048


def _tr_body(x_ref, o_ref):
    o_ref[...] = x_ref[...].T


def _transpose_idx(indices):
    return pl.pallas_call(
        _tr_body,
        grid=(B // _TR_BLK,),
        in_specs=[pl.BlockSpec((_TR_BLK, L), lambda i: (i, 0))],
        out_specs=pl.BlockSpec((L, _TR_BLK), lambda i: (0, i)),
        out_shape=jax.ShapeDtypeStruct((L, B), jnp.int32),
    )(indices)


def _mlp_body(x_ref, w1_ref, b1_ref, w2_ref, b2_ref, out_ref):
    x = x_ref[...] * jnp.float32(1.0 / L)
    h = jnp.dot(x, w1_ref[...], preferred_element_type=jnp.float32) + b1_ref[...]
    h = jnp.maximum(h, 0.0)
    out_ref[...] = jnp.dot(h, w2_ref[...], preferred_element_type=jnp.float32) + b2_ref[...]


def _mlp(pooled, W1, b1, W2, b2):
    return pl.pallas_call(
        _mlp_body,
        out_shape=jax.ShapeDtypeStruct((B, 1), jnp.float32),
    )(pooled, W1, b1.reshape(1, -1), W2, b2.reshape(1, -1))


def kernel(indices, table, W1, b1, W2, b2):
    idx_t = _transpose_idx(indices.astype(jnp.int32))
    pooled = _pool(idx_t, table)
    return _mlp(pooled, W1, b1, W2, b2)


# speedup vs baseline: 1.0449x; 1.0449x over previous
import functools

import jax
import jax.numpy as jnp
from jax import lax
from jax.experimental import pallas as pl
from jax.experimental.pallas import tpu as pltpu
from jax.experimental.pallas import tpu_sc as plsc

B = 16384
L = 200
D = 32
NW = 32
BPW = B // NW
CH = 40
NCH = L // CH


def _pool_body(idxt_hbm, table_hbm, pooled_hbm,
               idx_a, idx_b, acc0, acc1, acc2, acc3,
               sem_i, sem0, sem1, sem2, sem3):
    wid = lax.axis_index("s") * 2 + lax.axis_index("c")
    base = wid * BPW
    idx_bufs = (idx_a, idx_b)
    accs = (acc0, acc1, acc2, acc3)
    sems = (sem0, sem1, sem2, sem3)

    def idx_fetch(c, buf):
        return pltpu.async_copy(
            idxt_hbm.at[pl.ds(c * CH, CH), pl.ds(base, BPW)], buf, sem_i)

    idx_fetch(0, idx_a).wait()
    fetch1 = idx_fetch(1, idx_b)

    for a in range(4):
        pltpu.async_copy(table_hbm.at[idx_a.at[a]], accs[a], sems[a])

    def make_quad_body(idx_buf):
        def quad_body(k, _):
            for a in range(4):
                row = idx_buf.at[4 * k + a]
                pltpu.make_async_copy(table_hbm.at[row], accs[a], sems[a]).wait()
                pltpu.async_copy(table_hbm.at[row], accs[a], sems[a], add=True)
            return 0
        return quad_body

    lax.fori_loop(1, CH // 4, make_quad_body(idx_a), 0)

    pending = fetch1
    for c in range(1, NCH):
        buf = idx_bufs[c % 2]
        pending.wait()
        if c + 1 < NCH:
            pending = idx_fetch(c + 1, idx_bufs[(c + 1) % 2])
        lax.fori_loop(0, CH // 4, make_quad_body(buf), 0)

    for a in range(4):
        pltpu.make_async_copy(table_hbm.at[idx_a.at[0]], accs[a], sems[a]).wait()

    def comb_body(r, _):
        for h in (0, 16):
            acc0[r, pl.ds(h, 16)] = (
                (acc0[r, pl.ds(h, 16)] + acc1[r, pl.ds(h, 16)])
                + (acc2[r, pl.ds(h, 16)] + acc3[r, pl.ds(h, 16)])
            )
        return 0

    lax.fori_loop(0, BPW, comb_body, 0)
    pltpu.sync_copy(acc0, pooled_hbm.at[pl.ds(base, BPW)])


_pool = functools.partial(
    pl.kernel,
    mesh=plsc.VectorSubcoreMesh(core_axis_name="c", subcore_axis_name="s"),
    compiler_params=pltpu.CompilerParams(use_tc_tiling_on_sc=False),
    out_type=jax.ShapeDtypeStruct((B, D), jnp.float32),
    scratch_types=[
        pltpu.VMEM((CH, BPW), jnp.int32),
        pltpu.VMEM((CH, BPW), jnp.int32),
        pltpu.VMEM((BPW, D), jnp.float32),
        pltpu.VMEM((BPW, D), jnp.float32),
        pltpu.VMEM((BPW, D), jnp.float32),
        pltpu.VMEM((BPW, D), jnp.float32),
        pltpu.SemaphoreType.DMA,
        pltpu.SemaphoreType.DMA,
        pltpu.SemaphoreType.DMA,
        pltpu.SemaphoreType.DMA,
        pltpu.SemaphoreType.DMA,
    ],
)(_pool_body)


_TR_BLK = 2048


def _tr_body(x_ref, o_ref):
    o_ref[...] = x_ref[...].T


def _transpose_idx(indices):
    return pl.pallas_call(
        _tr_body,
        grid=(B // _TR_BLK,),
        in_specs=[pl.BlockSpec((_TR_BLK, L), lambda i: (i, 0))],
        out_specs=pl.BlockSpec((L, _TR_BLK), lambda i: (0, i)),
        out_shape=jax.ShapeDtypeStruct((L, B), jnp.int32),
    )(indices)


def _mlp_body(x_ref, w1_ref, b1_ref, w2_ref, b2_ref, out_ref):
    x = x_ref[...] * jnp.float32(1.0 / L)
    h = jnp.dot(x, w1_ref[...], preferred_element_type=jnp.float32) + b1_ref[...]
    h = jnp.maximum(h, 0.0)
    out_ref[...] = jnp.dot(h, w2_ref[...], preferred_element_type=jnp.float32) + b2_ref[...]


def _mlp(pooled, W1, b1, W2, b2):
    return pl.pallas_call(
        _mlp_body,
        out_shape=jax.ShapeDtypeStruct((B, 1), jnp.float32),
    )(pooled, W1, b1.reshape(1, -1), W2, b2.reshape(1, -1))


def kernel(indices, table, W1, b1, W2, b2):
    idx_t = _transpose_idx(indices.astype(jnp.int32))
    pooled = _pool(idx_t, table)
    return _mlp(pooled, W1, b1, W2, b2)


# speedup vs baseline: 1.0592x; 1.0137x over previous
import functools

import jax
import jax.numpy as jnp
from jax import lax
from jax.experimental import pallas as pl
from jax.experimental.pallas import tpu as pltpu
from jax.experimental.pallas import tpu_sc as plsc

B = 16384
L = 200
D = 32
NW = 32
BPW = B // NW
CH = 40
NCH = L // CH


def _pool_body(idxt_hbm, table_hbm, pooled_hbm,
               idx_a, idx_b, acc0, acc1, acc2, acc3,
               sem_i, sem0, sem1, sem2, sem3):
    wid = lax.axis_index("s") * 2 + lax.axis_index("c")
    base = wid * BPW
    idx_bufs = (idx_a, idx_b)
    accs = (acc0, acc1, acc2, acc3)
    sems = (sem0, sem1, sem2, sem3)

    def idx_fetch(c, buf):
        return pltpu.async_copy(
            idxt_hbm.at[pl.ds(wid * (L * BPW) + c * (CH * BPW), CH * BPW)],
            buf, sem_i)

    idx_fetch(0, idx_a).wait()
    fetch1 = idx_fetch(1, idx_b)

    for a in range(4):
        pltpu.async_copy(
            table_hbm.at[idx_a.at[pl.ds(a * BPW, BPW)]], accs[a], sems[a])

    def make_quad_body(idx_buf):
        def quad_body(k, _):
            for a in range(4):
                row = idx_buf.at[pl.ds((4 * k + a) * BPW, BPW)]
                pltpu.make_async_copy(table_hbm.at[row], accs[a], sems[a]).wait()
                pltpu.async_copy(table_hbm.at[row], accs[a], sems[a], add=True)
            return 0
        return quad_body

    lax.fori_loop(1, CH // 4, make_quad_body(idx_a), 0)

    pending = fetch1
    for c in range(1, NCH):
        buf = idx_bufs[c % 2]
        pending.wait()
        if c + 1 < NCH:
            pending = idx_fetch(c + 1, idx_bufs[(c + 1) % 2])
        lax.fori_loop(0, CH // 4, make_quad_body(buf), 0)

    for a in range(4):
        pltpu.make_async_copy(
            table_hbm.at[idx_a.at[pl.ds(0, BPW)]], accs[a], sems[a]).wait()

    def comb_body(r, _):
        for h in (0, 16):
            acc0[r, pl.ds(h, 16)] = (
                (acc0[r, pl.ds(h, 16)] + acc1[r, pl.ds(h, 16)])
                + (acc2[r, pl.ds(h, 16)] + acc3[r, pl.ds(h, 16)])
            )
        return 0

    lax.fori_loop(0, BPW, comb_body, 0)
    pltpu.sync_copy(acc0, pooled_hbm.at[pl.ds(base, BPW)])


_pool = functools.partial(
    pl.kernel,
    mesh=plsc.VectorSubcoreMesh(core_axis_name="c", subcore_axis_name="s"),
    compiler_params=pltpu.CompilerParams(use_tc_tiling_on_sc=False),
    out_type=jax.ShapeDtypeStruct((B, D), jnp.float32),
    scratch_types=[
        pltpu.VMEM((CH * BPW,), jnp.int32),
        pltpu.VMEM((CH * BPW,), jnp.int32),
        pltpu.VMEM((BPW, D), jnp.float32),
        pltpu.VMEM((BPW, D), jnp.float32),
        pltpu.VMEM((BPW, D), jnp.float32),
        pltpu.VMEM((BPW, D), jnp.float32),
        pltpu.SemaphoreType.DMA,
        pltpu.SemaphoreType.DMA,
        pltpu.SemaphoreType.DMA,
        pltpu.SemaphoreType.DMA,
        pltpu.SemaphoreType.DMA,
    ],
)(_pool_body)


def _perm_body(x_ref, o_ref):
    o_ref[...] = x_ref[...].T.reshape(-1)


def _permute_idx(indices):
    return pl.pallas_call(
        _perm_body,
        grid=(NW,),
        in_specs=[pl.BlockSpec((BPW, L), lambda i: (i, 0))],
        out_specs=pl.BlockSpec((L * BPW,), lambda i: (i,)),
        out_shape=jax.ShapeDtypeStruct((B * L,), jnp.int32),
    )(indices)


def _mlp_body(x_ref, w1_ref, b1_ref, w2_ref, b2_ref, out_ref):
    x = x_ref[...] * jnp.float32(1.0 / L)
    h = jnp.dot(x, w1_ref[...], preferred_element_type=jnp.float32) + b1_ref[...]
    h = jnp.maximum(h, 0.0)
    out_ref[...] = jnp.dot(h, w2_ref[...], preferred_element_type=jnp.float32) + b2_ref[...]


def _mlp(pooled, W1, b1, W2, b2):
    return pl.pallas_call(
        _mlp_body,
        out_shape=jax.ShapeDtypeStruct((B, 1), jnp.float32),
    )(pooled, W1, b1.reshape(1, -1), W2, b2.reshape(1, -1))


def kernel(indices, table, W1, b1, W2, b2):
    idx_t = _permute_idx(indices.astype(jnp.int32))
    pooled = _pool(idx_t, table)
    return _mlp(pooled, W1, b1, W2, b2)


# speedup vs baseline: 1.0710x; 1.0111x over previous
import functools

import jax
import jax.numpy as jnp
from jax import lax
from jax.experimental import pallas as pl
from jax.experimental.pallas import tpu as pltpu
from jax.experimental.pallas import tpu_sc as plsc

B = 16384
L = 200
D = 32
NW = 32
BPW = B // NW
NACC = 5
CH = 20
NCH = L // CH


def _pool_body(idxt_hbm, table_hbm, pooled_hbm,
               idx_a, idx_b, acc0, acc1, acc2, acc3, acc4,
               sem_i, sem0, sem1, sem2, sem3, sem4):
    wid = lax.axis_index("s") * 2 + lax.axis_index("c")
    base = wid * BPW
    idx_bufs = (idx_a, idx_b)
    accs = (acc0, acc1, acc2, acc3, acc4)
    sems = (sem0, sem1, sem2, sem3, sem4)

    def idx_fetch(c, buf):
        return pltpu.async_copy(
            idxt_hbm.at[pl.ds(wid * (L * BPW) + c * (CH * BPW), CH * BPW)],
            buf, sem_i)

    idx_fetch(0, idx_a).wait()
    fetch1 = idx_fetch(1, idx_b)

    for a in range(NACC):
        pltpu.async_copy(
            table_hbm.at[idx_a.at[pl.ds(a * BPW, BPW)]], accs[a], sems[a])

    def make_quad_body(idx_buf):
        def quad_body(k, _):
            for a in range(NACC):
                row = idx_buf.at[pl.ds((NACC * k + a) * BPW, BPW)]
                pltpu.make_async_copy(table_hbm.at[row], accs[a], sems[a]).wait()
                pltpu.async_copy(table_hbm.at[row], accs[a], sems[a], add=True)
            return 0
        return quad_body

    lax.fori_loop(1, CH // NACC, make_quad_body(idx_a), 0)

    pending = fetch1
    for c in range(1, NCH):
        buf = idx_bufs[c % 2]
        pending.wait()
        if c + 1 < NCH:
            pending = idx_fetch(c + 1, idx_bufs[(c + 1) % 2])
        lax.fori_loop(0, CH // NACC, make_quad_body(buf), 0)

    for a in range(NACC):
        pltpu.make_async_copy(
            table_hbm.at[idx_a.at[pl.ds(0, BPW)]], accs[a], sems[a]).wait()

    def comb_body(r, _):
        for h in (0, 16):
            acc0[r, pl.ds(h, 16)] = (
                (acc0[r, pl.ds(h, 16)] + acc1[r, pl.ds(h, 16)])
                + (acc2[r, pl.ds(h, 16)] + acc3[r, pl.ds(h, 16)])
            ) + acc4[r, pl.ds(h, 16)]
        return 0

    lax.fori_loop(0, BPW, comb_body, 0)
    pltpu.sync_copy(acc0, pooled_hbm.at[pl.ds(base, BPW)])


_pool = functools.partial(
    pl.kernel,
    mesh=plsc.VectorSubcoreMesh(core_axis_name="c", subcore_axis_name="s"),
    compiler_params=pltpu.CompilerParams(use_tc_tiling_on_sc=False),
    out_type=jax.ShapeDtypeStruct((B, D), jnp.float32),
    scratch_types=[
        pltpu.VMEM((CH * BPW,), jnp.int32),
        pltpu.VMEM((CH * BPW,), jnp.int32),
        pltpu.VMEM((BPW, D), jnp.float32),
        pltpu.VMEM((BPW, D), jnp.float32),
        pltpu.VMEM((BPW, D), jnp.float32),
        pltpu.VMEM((BPW, D), jnp.float32),
        pltpu.VMEM((BPW, D), jnp.float32),
        pltpu.SemaphoreType.DMA,
        pltpu.SemaphoreType.DMA,
        pltpu.SemaphoreType.DMA,
        pltpu.SemaphoreType.DMA,
        pltpu.SemaphoreType.DMA,
        pltpu.SemaphoreType.DMA,
    ],
)(_pool_body)


def _perm_body(x_ref, o_ref):
    o_ref[...] = x_ref[...].T.reshape(-1)


def _permute_idx(indices):
    return pl.pallas_call(
        _perm_body,
        grid=(NW,),
        in_specs=[pl.BlockSpec((BPW, L), lambda i: (i, 0))],
        out_specs=pl.BlockSpec((L * BPW,), lambda i: (i,)),
        out_shape=jax.ShapeDtypeStruct((B * L,), jnp.int32),
    )(indices)


def _mlp_body(x_ref, w1_ref, b1_ref, w2_ref, b2_ref, out_ref):
    x = x_ref[...] * jnp.float32(1.0 / L)
    h = jnp.dot(x, w1_ref[...], preferred_element_type=jnp.float32) + b1_ref[...]
    h = jnp.maximum(h, 0.0)
    out_ref[...] = jnp.dot(h, w2_ref[...], preferred_element_type=jnp.float32) + b2_ref[...]


def _mlp(pooled, W1, b1, W2, b2):
    return pl.pallas_call(
        _mlp_body,
        out_shape=jax.ShapeDtypeStruct((B, 1), jnp.float32),
    )(pooled, W1, b1.reshape(1, -1), W2, b2.reshape(1, -1))


def kernel(indices, table, W1, b1, W2, b2):
    idx_t = _permute_idx(indices.astype(jnp.int32))
    pooled = _pool(idx_t, table)
    return _mlp(pooled, W1, b1, W2, b2)


# speedup vs baseline: 1.0886x; 1.0164x over previous
import functools

import jax
import jax.numpy as jnp
from jax import lax
from jax.experimental import pallas as pl
from jax.experimental.pallas import tpu as pltpu
from jax.experimental.pallas import tpu_sc as plsc

B = 16384
L = 200
D = 32
NW = 32
BPW = B // NW
NACC = 5
CH = 20
NCH = L // CH


def _pool_body(idxt_hbm, table_hbm, pooled_hbm,
               idx_a, idx_b, acc0, acc1, acc2, acc3, acc4, out_v,
               sem_i, sem0, sem1, sem2, sem3, sem4):
    wid = lax.axis_index("s") * 2 + lax.axis_index("c")
    base = wid * BPW
    idx_bufs = (idx_a, idx_b)
    accs = (acc0, acc1, acc2, acc3, acc4)
    sems = (sem0, sem1, sem2, sem3, sem4)

    def idx_fetch(c, buf):
        return pltpu.async_copy(
            idxt_hbm.at[pl.ds(wid * (L * BPW) + c * (CH * BPW), CH * BPW)],
            buf, sem_i)

    idx_fetch(0, idx_a).wait()
    fetch1 = idx_fetch(1, idx_b)

    for a in range(NACC):
        pltpu.async_copy(
            table_hbm.at[idx_a.at[pl.ds(a * BPW, BPW)]], accs[a], sems[a])

    def make_quad_body(idx_buf):
        def quad_body(k, _):
            for a in range(NACC):
                row = idx_buf.at[pl.ds((NACC * k + a) * BPW, BPW)]
                pltpu.make_async_copy(table_hbm.at[row], accs[a], sems[a]).wait()
                pltpu.async_copy(table_hbm.at[row], accs[a], sems[a], add=True)
            return 0
        return quad_body

    lax.fori_loop(1, CH // NACC, make_quad_body(idx_a), 0)

    pending = fetch1
    for c in range(1, NCH):
        buf = idx_bufs[c % 2]
        pending.wait()
        if c + 1 < NCH:
            pending = idx_fetch(c + 1, idx_bufs[(c + 1) % 2])
        lax.fori_loop(0, CH // NACC, make_quad_body(buf), 0)

    for a in range(NACC):
        pltpu.make_async_copy(
            table_hbm.at[idx_a.at[pl.ds(0, BPW)]], accs[a], sems[a]).wait()

    def comb_body(q, _):
        for k in range(4):
            for h in (0, 16):
                out_v[q, pl.ds(k * D + h, 16)] = (
                    (acc0[4 * q + k, pl.ds(h, 16)] + acc1[4 * q + k, pl.ds(h, 16)])
                    + (acc2[4 * q + k, pl.ds(h, 16)] + acc3[4 * q + k, pl.ds(h, 16)])
                ) + acc4[4 * q + k, pl.ds(h, 16)]
        return 0

    lax.fori_loop(0, BPW // 4, comb_body, 0)
    pltpu.sync_copy(out_v, pooled_hbm.at[pl.ds(wid * (BPW // 4), BPW // 4)])


_pool = functools.partial(
    pl.kernel,
    mesh=plsc.VectorSubcoreMesh(core_axis_name="c", subcore_axis_name="s"),
    compiler_params=pltpu.CompilerParams(use_tc_tiling_on_sc=False),
    out_type=jax.ShapeDtypeStruct((B // 4, 4 * D), jnp.float32),
    scratch_types=[
        pltpu.VMEM((CH * BPW,), jnp.int32),
        pltpu.VMEM((CH * BPW,), jnp.int32),
        pltpu.VMEM((BPW, D), jnp.float32),
        pltpu.VMEM((BPW, D), jnp.float32),
        pltpu.VMEM((BPW, D), jnp.float32),
        pltpu.VMEM((BPW, D), jnp.float32),
        pltpu.VMEM((BPW, D), jnp.float32),
        pltpu.VMEM((BPW // 4, 4 * D), jnp.float32),
        pltpu.SemaphoreType.DMA,
        pltpu.SemaphoreType.DMA,
        pltpu.SemaphoreType.DMA,
        pltpu.SemaphoreType.DMA,
        pltpu.SemaphoreType.DMA,
        pltpu.SemaphoreType.DMA,
    ],
)(_pool_body)


def _perm_body(x_ref, o_ref):
    o_ref[...] = x_ref[...].T.reshape(-1)


def _permute_idx(indices):
    return pl.pallas_call(
        _perm_body,
        grid=(NW,),
        in_specs=[pl.BlockSpec((BPW, L), lambda i: (i, 0))],
        out_specs=pl.BlockSpec((L * BPW,), lambda i: (i,)),
        out_shape=jax.ShapeDtypeStruct((B * L,), jnp.int32),
    )(indices)


def _mlp_body(x_ref, w1_ref, b1_ref, w2_ref, b2_ref, out_ref):
    x = x_ref[...] * jnp.float32(1.0 / L)
    outs = []
    for k in range(4):
        h = jnp.dot(x[:, k * D:(k + 1) * D], w1_ref[...],
                    preferred_element_type=jnp.float32) + b1_ref[...]
        h = jnp.maximum(h, 0.0)
        outs.append(jnp.dot(h, w2_ref[...], preferred_element_type=jnp.float32)
                    + b2_ref[...])
    out_ref[...] = jnp.concatenate(outs, axis=1)


def _mlp(pooled_p, W1, b1, W2, b2):
    return pl.pallas_call(
        _mlp_body,
        out_shape=jax.ShapeDtypeStruct((B // 4, 4), jnp.float32),
    )(pooled_p, W1, b1.reshape(1, -1), W2, b2.reshape(1, -1))


def kernel(indices, table, W1, b1, W2, b2):
    idx_t = _permute_idx(indices.astype(jnp.int32))
    pooled_p = _pool(idx_t, table)
    return _mlp(pooled_p, W1, b1, W2, b2).reshape(B, 1)
